# baseline (speedup 1.0000x reference)
import functools

import jax
import jax.numpy as jnp
import numpy as np
from jax import lax
from jax.experimental import pallas as pl
from jax.experimental.pallas import tpu as pltpu
from jax.experimental.pallas import tpu_sc as plsc

K = 64
HIST = 200
NW = 16
VALID = 25
PASS = 4


def _din_body(idx_hbm, ut_hbm, mt_hbm, ct_hbm,
              w0_hbm, b0_hbm, w1_hbm, b1_hbm, w2_hbm, b2_hbm,
              out_hbm,
              idx_v, blks, abuf, stage_v, shared, part_v,
              w0_v, b0_v, w1_v, b1_v, w2_v, b2_v,
              res_v, gsem, absem):
    cid = lax.axis_index("c")
    sid = lax.axis_index("s")

    @pl.when(cid == 0)
    def _core0():
        pltpu.sync_copy(idx_hbm.at[sid], idx_v)
        i0 = idx_v[pl.ds(0, 16)]
        i1 = idx_v[pl.ds(16, 16)]
        lane16 = lax.iota(jnp.int32, 16)
        cm0 = i0 & (-16)
        cm1 = i1 & (-16)
        l0 = i0 & 15
        l1 = i1 & 15
        cms = [cm0[l] for l in range(16)] + [cm1[l] for l in range(VALID - 16)]
        lns = [l0[l] for l in range(16)] + [l1[l] for l in range(VALID - 16)]
        base = [pl.multiple_of(c & (-128), 128) for c in cms]
        off = [cms[j] - (cms[j] & (-128)) for j in range(VALID)]

        i2 = idx_v[pl.ds(32, 16)]
        ab_id = jnp.where(sid == 1, i2[0], i2[1])
        ab_base = pl.multiple_of(ab_id & (-128), 128)
        ab_off = (ab_id & (-16)) - (ab_id & (-128))

        @pl.when(sid == 1)
        def _fetch_a():
            pltpu.async_copy(ut_hbm.at[:, pl.ds(ab_base, 128)], abuf, absem)

        @pl.when(sid == 2)
        def _fetch_b():
            pltpu.async_copy(mt_hbm.at[:, pl.ds(ab_base, 128)], abuf, absem)

        @pl.when(sid == 0)
        def _prefetch_w():
            pltpu.sync_copy(w0_hbm, w0_v)
            pltpu.sync_copy(b0_hbm, b0_v)
            pltpu.sync_copy(w1_hbm, w1_v)
            pltpu.sync_copy(b1_hbm, b1_v)
            pltpu.sync_copy(w2_hbm, w2_v)
            pltpu.sync_copy(b2_hbm, b2_v)

        chunks = [jnp.zeros((16,), jnp.float32) for _ in range(4)]
        fmasks = [lane16 == m for m in range(16)]

        def start(j):
            return pltpu.async_copy(
                ct_hbm.at[:, pl.ds(base[j], 128)],
                blks.at[j % PASS], gsem)

        cps = [start(j) for j in range(PASS)]
        for j in range(VALID):
            cps[j].wait()
            lvec = jnp.zeros((16,), jnp.int32) + lns[j]
            for f in range(K):
                v = blks[j % PASS, f, pl.ds(off[j], 16)]
                val = v[lvec]
                i = f // 16
                chunks[i] = chunks[i] + jnp.where(fmasks[f - 16 * i], val, 0.0)
            if j + PASS < VALID:
                cps.append(start(j + PASS))
        for i in range(4):
            stage_v[pl.ds(16 * i, 16)] = chunks[i]
        pltpu.sync_copy(stage_v, shared.at[sid])

        @pl.when((sid == 1) | (sid == 2))
        def _extract_ab():
            pltpu.make_async_copy(ut_hbm.at[:, pl.ds(ab_base, 128)],
                                  abuf, absem).wait()
            ch = [jnp.zeros((16,), jnp.float32) for _ in range(4)]
            ab_lvec = jnp.zeros((16,), jnp.int32) + (ab_id & 15)
            for f in range(K):
                v = abuf[f, pl.ds(ab_off, 16)]
                val = v[ab_lvec]
                i = f // 16
                ch[i] = ch[i] + jnp.where(lane16 == (f - 16 * i), val, 0.0)
            for i in range(4):
                stage_v[pl.ds(16 * i, 16)] = ch[i]
            pltpu.sync_copy(stage_v, shared.at[15 + sid])

        plsc.subcore_barrier()

        @pl.when(sid == 0)
        def _finish():
            pltpu.sync_copy(shared, part_v)
            inv = 1.0 / float(HIST)
            fea = []
            for i in range(4):
                fea.append(part_v[16, pl.ds(16 * i, 16)])
            for i in range(4):
                fea.append(part_v[17, pl.ds(16 * i, 16)])
            for seg in range(2):
                for i in range(4):
                    s = part_v[8 * seg, pl.ds(16 * i, 16)]
                    for j in range(1, 8):
                        s = s + part_v[8 * seg + j, pl.ds(16 * i, 16)]
                    fea.append(s * inv)

            h0 = b0_v[pl.ds(0, 16)]
            h1 = b0_v[pl.ds(16, 16)]
            for t in range(16):
                chunk = fea[t]
                for l in range(16):
                    f = chunk[l]
                    k = 16 * t + l
                    h0 = h0 + f * w0_v[pl.ds(32 * k, 16)]
                    h1 = h1 + f * w0_v[pl.ds(32 * k + 16, 16)]
            h0 = jnp.maximum(h0, 0.0)
            h1 = jnp.maximum(h1, 0.0)

            g = b1_v[pl.ds(0, 16)]
            for k in range(16):
                g = g + h0[k] * w1_v[pl.ds(16 * k, 16)]
            for k in range(4):
                g = g + h1[k] * w1_v[pl.ds(16 * (16 + k), 16)]
            g = jnp.maximum(g, 0.0)

            r = b2_v[pl.ds(0, 16)]
            for k in range(8):
                r = r + g[k] * w2_v[pl.ds(16 * k, 16)]
            delta = r[0] - r[1]
            dvec = jnp.zeros((16,), jnp.float32) + delta
            res_v[...] = 1.0 / (1.0 + jnp.exp(dvec))
            pltpu.sync_copy(res_v, out_hbm)


@jax.jit
def _din_sc(idx2d, ut_t, mt_t, ct_t, w0p, b0p, w1p, b1p, w2p, b2p):
    mesh = plsc.VectorSubcoreMesh(core_axis_name="c", subcore_axis_name="s")
    f32 = jnp.float32
    run = functools.partial(
        pl.kernel,
        out_type=jax.ShapeDtypeStruct((16,), f32),
        mesh=mesh,
        scratch_types=[
            pltpu.VMEM((128,), jnp.int32),
            pltpu.VMEM((PASS, K, 128), f32),
            pltpu.VMEM((K, 128), f32),
            pltpu.VMEM((K,), f32),
            pltpu.VMEM_SHARED((NW + 2, K), f32),
            pltpu.VMEM((NW + 2, K), f32),
            pltpu.VMEM((256 * 32,), f32),
            pltpu.VMEM((32,), f32),
            pltpu.VMEM((20 * 16,), f32),
            pltpu.VMEM((16,), f32),
            pltpu.VMEM((8 * 16,), f32),
            pltpu.VMEM((16,), f32),
            pltpu.VMEM((16,), f32),
            pltpu.SemaphoreType.DMA,
            pltpu.SemaphoreType.DMA,
        ],
        compiler_params=pltpu.CompilerParams(use_tc_tiling_on_sc=True),
    )(_din_body)
    return run(idx2d, ut_t, mt_t, ct_t, w0p, b0p, w1p, b1p, w2p, b2p)


def kernel(user_id, movie_id, movie_cate, user_rate, user_table, movie_table,
           movie_cate_table, W0, b0, W1, b1, W2, b2):
    i32 = jnp.int32
    cat = jnp.concatenate([movie_cate, user_rate]).astype(i32).reshape(NW, VALID)
    idx2d = jnp.pad(cat, ((0, 0), (0, 128 - VALID)))
    um = jnp.concatenate([user_id.astype(i32), movie_id.astype(i32)])
    idx2d = lax.dynamic_update_slice(
        idx2d, jnp.broadcast_to(um, (NW, 2)), (0, 32))
    w0p = jnp.pad(W0, ((0, 0), (0, 12))).reshape(-1)
    b0p = jnp.pad(b0, (0, 12))
    w1p = jnp.pad(W1, ((0, 0), (0, 8))).reshape(-1)
    b1p = jnp.pad(b1, (0, 8))
    w2p = jnp.pad(W2, ((0, 0), (0, 14))).reshape(-1)
    b2p = jnp.pad(b2, (0, 14))
    out = _din_sc(idx2d, user_table.T, movie_table.T, movie_cate_table.T,
                  w0p, b0p, w1p, b1p, w2p, b2p)
    return out[1:2]


# speedup vs baseline: 1.1020x; 1.1020x over previous
import functools

import jax
import jax.numpy as jnp
import numpy as np
from jax import lax
from jax.experimental import pallas as pl
from jax.experimental.pallas import tpu as pltpu
from jax.experimental.pallas import tpu_sc as plsc

K = 64
HIST = 200
NW = 16
VALID = 25
PASS = 4


def _din_body(idx_hbm, ut_hbm, mt_hbm, ct_hbm, wall_hbm,
              out_hbm,
              idx_v, blks, abuf, stage_v, shared, part_v,
              wall_v, res_v, gsem, absem):
    cid = lax.axis_index("c")
    sid = lax.axis_index("s")

    @pl.when(cid == 0)
    def _core0():
        pltpu.sync_copy(idx_hbm.at[sid], idx_v)
        i0 = idx_v[pl.ds(0, 16)]
        i1 = idx_v[pl.ds(16, 16)]
        lane16 = lax.iota(jnp.int32, 16)
        cm0 = i0 & (-16)
        cm1 = i1 & (-16)
        l0 = i0 & 15
        l1 = i1 & 15
        cms = [cm0[l] for l in range(16)] + [cm1[l] for l in range(VALID - 16)]
        lns = [l0[l] for l in range(16)] + [l1[l] for l in range(VALID - 16)]
        base = [pl.multiple_of(c & (-128), 128) for c in cms]
        off = [cms[j] - (cms[j] & (-128)) for j in range(VALID)]

        i2 = idx_v[pl.ds(32, 16)]
        ab_id = jnp.where(sid == 1, i2[0], i2[1])
        ab_base = pl.multiple_of(ab_id & (-128), 128)
        ab_off = (ab_id & (-16)) - (ab_id & (-128))

        @pl.when(sid == 1)
        def _fetch_a():
            pltpu.async_copy(ut_hbm.at[:, pl.ds(ab_base, 128)], abuf, absem)

        @pl.when(sid == 2)
        def _fetch_b():
            pltpu.async_copy(mt_hbm.at[:, pl.ds(ab_base, 128)], abuf, absem)

        @pl.when(sid == 0)
        def _prefetch_w():
            pltpu.sync_copy(wall_hbm, wall_v)

        chunks = [jnp.zeros((16,), jnp.float32) for _ in range(4)]
        fmasks = [lane16 == m for m in range(16)]

        def start(j):
            return pltpu.async_copy(
                ct_hbm.at[:, pl.ds(base[j], 128)],
                blks.at[j % PASS], gsem)

        cps = [start(j) for j in range(PASS)]
        for j in range(VALID):
            cps[j].wait()
            lvec = jnp.zeros((16,), jnp.int32) + lns[j]
            for f in range(K):
                v = blks[j % PASS, f, pl.ds(off[j], 16)]
                val = v[lvec]
                i = f // 16
                chunks[i] = chunks[i] + jnp.where(fmasks[f - 16 * i], val, 0.0)
            if j + PASS < VALID:
                cps.append(start(j + PASS))
        for i in range(4):
            stage_v[pl.ds(16 * i, 16)] = chunks[i]
        pltpu.sync_copy(stage_v, shared.at[sid])

        @pl.when((sid == 1) | (sid == 2))
        def _extract_ab():
            pltpu.make_async_copy(ut_hbm.at[:, pl.ds(ab_base, 128)],
                                  abuf, absem).wait()
            ch = [jnp.zeros((16,), jnp.float32) for _ in range(4)]
            ab_lvec = jnp.zeros((16,), jnp.int32) + (ab_id & 15)
            for f in range(K):
                v = abuf[f, pl.ds(ab_off, 16)]
                val = v[ab_lvec]
                i = f // 16
                ch[i] = ch[i] + jnp.where(lane16 == (f - 16 * i), val, 0.0)
            for i in range(4):
                stage_v[pl.ds(16 * i, 16)] = ch[i]
            pltpu.sync_copy(stage_v, shared.at[15 + sid])

        plsc.subcore_barrier()

        @pl.when(sid == 0)
        def _finish():
            pltpu.sync_copy(shared, part_v)
            inv = 1.0 / float(HIST)
            fea = []
            for i in range(4):
                fea.append(part_v[16, pl.ds(16 * i, 16)])
            for i in range(4):
                fea.append(part_v[17, pl.ds(16 * i, 16)])
            for seg in range(2):
                for i in range(4):
                    s = part_v[8 * seg, pl.ds(16 * i, 16)]
                    for j in range(1, 8):
                        s = s + part_v[8 * seg + j, pl.ds(16 * i, 16)]
                    fea.append(s * inv)

            h0 = wall_v[pl.ds(8640, 16)]
            h1 = wall_v[pl.ds(8656, 16)]
            for t in range(16):
                chunk = fea[t]
                for l in range(16):
                    f = chunk[l]
                    k = 16 * t + l
                    h0 = h0 + f * wall_v[pl.ds(32 * k, 16)]
                    h1 = h1 + f * wall_v[pl.ds(32 * k + 16, 16)]
            h0 = jnp.maximum(h0, 0.0)
            h1 = jnp.maximum(h1, 0.0)

            g = wall_v[pl.ds(8672, 16)]
            for k in range(16):
                g = g + h0[k] * wall_v[pl.ds(8192 + 16 * k, 16)]
            for k in range(4):
                g = g + h1[k] * wall_v[pl.ds(8192 + 16 * (16 + k), 16)]
            g = jnp.maximum(g, 0.0)

            r = wall_v[pl.ds(8688, 16)]
            for k in range(8):
                r = r + g[k] * wall_v[pl.ds(8512 + 16 * k, 16)]
            delta = r[0] - r[1]
            dvec = jnp.zeros((16,), jnp.float32) + delta
            res_v[...] = 1.0 / (1.0 + jnp.exp(dvec))
            pltpu.sync_copy(res_v, out_hbm)


@jax.jit
def _din_sc(idx2d, ut_t, mt_t, ct_t, wall):
    mesh = plsc.VectorSubcoreMesh(core_axis_name="c", subcore_axis_name="s")
    f32 = jnp.float32
    run = functools.partial(
        pl.kernel,
        out_type=jax.ShapeDtypeStruct((16,), f32),
        mesh=mesh,
        scratch_types=[
            pltpu.VMEM((128,), jnp.int32),
            pltpu.VMEM((PASS, K, 128), f32),
            pltpu.VMEM((K, 128), f32),
            pltpu.VMEM((K,), f32),
            pltpu.VMEM_SHARED((NW + 2, K), f32),
            pltpu.VMEM((NW + 2, K), f32),
            pltpu.VMEM((544 * 16,), f32),
            pltpu.VMEM((16,), f32),
            pltpu.SemaphoreType.DMA,
            pltpu.SemaphoreType.DMA,
        ],
        compiler_params=pltpu.CompilerParams(use_tc_tiling_on_sc=True),
    )(_din_body)
    return run(idx2d, ut_t, mt_t, ct_t, wall)


def kernel(user_id, movie_id, movie_cate, user_rate, user_table, movie_table,
           movie_cate_table, W0, b0, W1, b1, W2, b2):
    i32 = jnp.int32
    cat = jnp.concatenate([movie_cate, user_rate]).astype(i32).reshape(NW, VALID)
    idx2d = jnp.pad(cat, ((0, 0), (0, 128 - VALID)))
    um = jnp.concatenate([user_id.astype(i32), movie_id.astype(i32)])
    idx2d = lax.dynamic_update_slice(
        idx2d, jnp.broadcast_to(um, (NW, 2)), (0, 32))
    wall = jnp.concatenate([
        jnp.pad(W0, ((0, 0), (0, 12))).reshape(512, 16),
        jnp.pad(W1, ((0, 0), (0, 8))),
        jnp.pad(W2, ((0, 0), (0, 14))),
        jnp.pad(b0, (0, 12)).reshape(2, 16),
        jnp.pad(b1, (0, 8)).reshape(1, 16),
        jnp.pad(b2, (0, 14)).reshape(1, 16),
    ]).reshape(-1)
    out = _din_sc(idx2d, user_table.T, movie_table.T, movie_cate_table.T,
                  wall)
    return out[1:2]


# speedup vs baseline: 1.1442x; 1.0383x over previous
import functools

import jax
import jax.numpy as jnp
import numpy as np
from jax import lax
from jax.experimental import pallas as pl
from jax.experimental.pallas import tpu as pltpu
from jax.experimental.pallas import tpu_sc as plsc

K = 64
HIST = 200
NW = 16
VALID = 25
PASS = 8


def _din_body(idx_hbm, ut_hbm, mt_hbm, ct_hbm, wall_hbm,
              out_hbm,
              idx_v, blks, abuf, stage_v, shared, part_v,
              wall_v, res_v, gsem, absem):
    cid = lax.axis_index("c")
    sid = lax.axis_index("s")

    @pl.when(cid == 0)
    def _core0():
        pltpu.sync_copy(idx_hbm.at[sid], idx_v)
        i0 = idx_v[pl.ds(0, 16)]
        i1 = idx_v[pl.ds(16, 16)]
        lane16 = lax.iota(jnp.int32, 16)
        cm0 = i0 & (-16)
        cm1 = i1 & (-16)
        l0 = i0 & 15
        l1 = i1 & 15
        cms = [cm0[l] for l in range(16)] + [cm1[l] for l in range(VALID - 16)]
        lns = [l0[l] for l in range(16)] + [l1[l] for l in range(VALID - 16)]
        base = [pl.multiple_of(c & (-128), 128) for c in cms]
        off = [cms[j] - (cms[j] & (-128)) for j in range(VALID)]

        i2 = idx_v[pl.ds(32, 16)]
        ab_id = jnp.where(sid == 1, i2[0], i2[1])
        ab_base = pl.multiple_of(ab_id & (-128), 128)
        ab_off = (ab_id & (-16)) - (ab_id & (-128))

        @pl.when(sid == 1)
        def _fetch_a():
            pltpu.async_copy(ut_hbm.at[:, pl.ds(ab_base, 128)], abuf, absem)

        @pl.when(sid == 2)
        def _fetch_b():
            pltpu.async_copy(mt_hbm.at[:, pl.ds(ab_base, 128)], abuf, absem)

        @pl.when(sid == 0)
        def _prefetch_w():
            pltpu.sync_copy(wall_hbm, wall_v)

        chunks = [jnp.zeros((16,), jnp.float32) for _ in range(4)]
        fmasks = [lane16 == m for m in range(16)]

        def start(j):
            return pltpu.async_copy(
                ct_hbm.at[:, pl.ds(base[j], 128)],
                blks.at[j % PASS], gsem)

        cps = [start(j) for j in range(PASS)]
        for j in range(VALID):
            cps[j].wait()
            lvec = jnp.zeros((16,), jnp.int32) + lns[j]
            for f in range(K):
                v = blks[j % PASS, f, pl.ds(off[j], 16)]
                val = v[lvec]
                i = f // 16
                chunks[i] = chunks[i] + jnp.where(fmasks[f - 16 * i], val, 0.0)
            if j + PASS < VALID:
                cps.append(start(j + PASS))
        for i in range(4):
            stage_v[pl.ds(16 * i, 16)] = chunks[i]
        pltpu.sync_copy(stage_v, shared.at[sid])

        @pl.when((sid == 1) | (sid == 2))
        def _extract_ab():
            pltpu.make_async_copy(ut_hbm.at[:, pl.ds(ab_base, 128)],
                                  abuf, absem).wait()
            ch = [jnp.zeros((16,), jnp.float32) for _ in range(4)]
            ab_lvec = jnp.zeros((16,), jnp.int32) + (ab_id & 15)
            for f in range(K):
                v = abuf[f, pl.ds(ab_off, 16)]
                val = v[ab_lvec]
                i = f // 16
                ch[i] = ch[i] + jnp.where(lane16 == (f - 16 * i), val, 0.0)
            for i in range(4):
                stage_v[pl.ds(16 * i, 16)] = ch[i]
            pltpu.sync_copy(stage_v, shared.at[15 + sid])

        plsc.subcore_barrier()

        @pl.when(sid == 0)
        def _finish():
            pltpu.sync_copy(shared, part_v)
            inv = 1.0 / float(HIST)
            fea = []
            for i in range(4):
                fea.append(part_v[16, pl.ds(16 * i, 16)])
            for i in range(4):
                fea.append(part_v[17, pl.ds(16 * i, 16)])
            for seg in range(2):
                for i in range(4):
                    s = part_v[8 * seg, pl.ds(16 * i, 16)]
                    for j in range(1, 8):
                        s = s + part_v[8 * seg + j, pl.ds(16 * i, 16)]
                    fea.append(s * inv)

            h0 = wall_v[pl.ds(8640, 16)]
            h1 = wall_v[pl.ds(8656, 16)]
            for t in range(16):
                chunk = fea[t]
                for l in range(16):
                    f = chunk[l]
                    k = 16 * t + l
                    h0 = h0 + f * wall_v[pl.ds(32 * k, 16)]
                    h1 = h1 + f * wall_v[pl.ds(32 * k + 16, 16)]
            h0 = jnp.maximum(h0, 0.0)
            h1 = jnp.maximum(h1, 0.0)

            g = wall_v[pl.ds(8672, 16)]
            for k in range(16):
                g = g + h0[k] * wall_v[pl.ds(8192 + 16 * k, 16)]
            for k in range(4):
                g = g + h1[k] * wall_v[pl.ds(8192 + 16 * (16 + k), 16)]
            g = jnp.maximum(g, 0.0)

            r = wall_v[pl.ds(8688, 16)]
            for k in range(8):
                r = r + g[k] * wall_v[pl.ds(8512 + 16 * k, 16)]
            delta = r[0] - r[1]
            dvec = jnp.zeros((16,), jnp.float32) + delta
            res_v[...] = 1.0 / (1.0 + jnp.exp(dvec))
            pltpu.sync_copy(res_v, out_hbm)


@jax.jit
def _din_sc(idx2d, ut_t, mt_t, ct_t, wall):
    mesh = plsc.VectorSubcoreMesh(core_axis_name="c", subcore_axis_name="s")
    f32 = jnp.float32
    run = functools.partial(
        pl.kernel,
        out_type=jax.ShapeDtypeStruct((16,), f32),
        mesh=mesh,
        scratch_types=[
            pltpu.VMEM((128,), jnp.int32),
            pltpu.VMEM((PASS, K, 128), f32),
            pltpu.VMEM((K, 128), f32),
            pltpu.VMEM((K,), f32),
            pltpu.VMEM_SHARED((NW + 2, K), f32),
            pltpu.VMEM((NW + 2, K), f32),
            pltpu.VMEM((544 * 16,), f32),
            pltpu.VMEM((16,), f32),
            pltpu.SemaphoreType.DMA,
            pltpu.SemaphoreType.DMA,
        ],
        compiler_params=pltpu.CompilerParams(use_tc_tiling_on_sc=True),
    )(_din_body)
    return run(idx2d, ut_t, mt_t, ct_t, wall)


def kernel(user_id, movie_id, movie_cate, user_rate, user_table, movie_table,
           movie_cate_table, W0, b0, W1, b1, W2, b2):
    i32 = jnp.int32
    cat = jnp.concatenate([movie_cate, user_rate]).astype(i32).reshape(NW, VALID)
    idx2d = jnp.pad(cat, ((0, 0), (0, 128 - VALID)))
    um = jnp.concatenate([user_id.astype(i32), movie_id.astype(i32)])
    idx2d = lax.dynamic_update_slice(
        idx2d, jnp.broadcast_to(um, (NW, 2)), (0, 32))
    wall = jnp.concatenate([
        jnp.pad(W0, ((0, 0), (0, 12))).reshape(512, 16),
        jnp.pad(W1, ((0, 0), (0, 8))),
        jnp.pad(W2, ((0, 0), (0, 14))),
        jnp.pad(b0, (0, 12)).reshape(2, 16),
        jnp.pad(b1, (0, 8)).reshape(1, 16),
        jnp.pad(b2, (0, 14)).reshape(1, 16),
    ]).reshape(-1)
    out = _din_sc(idx2d, user_table.T, movie_table.T, movie_cate_table.T,
                  wall)
    return out[1:2]


# speedup vs baseline: 1.1687x; 1.0214x over previous
import functools

import jax
import jax.numpy as jnp
import numpy as np
from jax import lax
from jax.experimental import pallas as pl
from jax.experimental.pallas import tpu as pltpu
from jax.experimental.pallas import tpu_sc as plsc

K = 64
HIST = 200
NW = 16
VALID = 25
PASS = 10


def _din_body(idx_hbm, ut_hbm, mt_hbm, ct_hbm, wall_hbm,
              out_hbm,
              idx_v, blks, abuf, stage_v, shared, part_v,
              wall_v, res_v, gsem, absem):
    cid = lax.axis_index("c")
    sid = lax.axis_index("s")

    @pl.when(cid == 0)
    def _core0():
        pltpu.sync_copy(idx_hbm.at[sid], idx_v)
        i0 = idx_v[pl.ds(0, 16)]
        i1 = idx_v[pl.ds(16, 16)]
        lane16 = lax.iota(jnp.int32, 16)
        cm0 = i0 & (-16)
        cm1 = i1 & (-16)
        l0 = i0 & 15
        l1 = i1 & 15
        cms = [cm0[l] for l in range(16)] + [cm1[l] for l in range(VALID - 16)]
        lns = [l0[l] for l in range(16)] + [l1[l] for l in range(VALID - 16)]
        base = [pl.multiple_of(c & (-128), 128) for c in cms]
        off = [cms[j] - (cms[j] & (-128)) for j in range(VALID)]

        i2 = idx_v[pl.ds(32, 16)]
        ab_id = jnp.where(sid == 1, i2[0], i2[1])
        ab_base = pl.multiple_of(ab_id & (-128), 128)
        ab_off = (ab_id & (-16)) - (ab_id & (-128))

        @pl.when(sid == 1)
        def _fetch_a():
            pltpu.async_copy(ut_hbm.at[:, pl.ds(ab_base, 128)], abuf, absem)

        @pl.when(sid == 2)
        def _fetch_b():
            pltpu.async_copy(mt_hbm.at[:, pl.ds(ab_base, 128)], abuf, absem)

        @pl.when(sid == 0)
        def _prefetch_w():
            pltpu.sync_copy(wall_hbm, wall_v)

        chunks = [jnp.zeros((16,), jnp.float32) for _ in range(4)]
        fmasks = [lane16 == m for m in range(16)]

        def start(j):
            return pltpu.async_copy(
                ct_hbm.at[:, pl.ds(base[j], 128)],
                blks.at[j % PASS], gsem)

        cps = [start(j) for j in range(PASS)]
        for j in range(VALID):
            cps[j].wait()
            lvec = jnp.zeros((16,), jnp.int32) + lns[j]
            for f in range(K):
                v = blks[j % PASS, f, pl.ds(off[j], 16)]
                val = v[lvec]
                i = f // 16
                chunks[i] = chunks[i] + jnp.where(fmasks[f - 16 * i], val, 0.0)
            if j + PASS < VALID:
                cps.append(start(j + PASS))
        for i in range(4):
            stage_v[pl.ds(16 * i, 16)] = chunks[i]
        pltpu.sync_copy(stage_v, shared.at[sid])

        @pl.when((sid == 1) | (sid == 2))
        def _extract_ab():
            pltpu.make_async_copy(ut_hbm.at[:, pl.ds(ab_base, 128)],
                                  abuf, absem).wait()
            ch = [jnp.zeros((16,), jnp.float32) for _ in range(4)]
            ab_lvec = jnp.zeros((16,), jnp.int32) + (ab_id & 15)
            for f in range(K):
                v = abuf[f, pl.ds(ab_off, 16)]
                val = v[ab_lvec]
                i = f // 16
                ch[i] = ch[i] + jnp.where(lane16 == (f - 16 * i), val, 0.0)
            for i in range(4):
                stage_v[pl.ds(16 * i, 16)] = ch[i]
            pltpu.sync_copy(stage_v, shared.at[15 + sid])

        plsc.subcore_barrier()

        @pl.when(sid == 0)
        def _finish():
            pltpu.sync_copy(shared, part_v)
            inv = 1.0 / float(HIST)
            fea = []
            for i in range(4):
                fea.append(part_v[16, pl.ds(16 * i, 16)])
            for i in range(4):
                fea.append(part_v[17, pl.ds(16 * i, 16)])
            for seg in range(2):
                for i in range(4):
                    s = part_v[8 * seg, pl.ds(16 * i, 16)]
                    for j in range(1, 8):
                        s = s + part_v[8 * seg + j, pl.ds(16 * i, 16)]
                    fea.append(s * inv)

            h0 = wall_v[pl.ds(8640, 16)]
            h1 = wall_v[pl.ds(8656, 16)]
            for t in range(16):
                chunk = fea[t]
                for l in range(16):
                    f = chunk[l]
                    k = 16 * t + l
                    h0 = h0 + f * wall_v[pl.ds(32 * k, 16)]
                    h1 = h1 + f * wall_v[pl.ds(32 * k + 16, 16)]
            h0 = jnp.maximum(h0, 0.0)
            h1 = jnp.maximum(h1, 0.0)

            g = wall_v[pl.ds(8672, 16)]
            for k in range(16):
                g = g + h0[k] * wall_v[pl.ds(8192 + 16 * k, 16)]
            for k in range(4):
                g = g + h1[k] * wall_v[pl.ds(8192 + 16 * (16 + k), 16)]
            g = jnp.maximum(g, 0.0)

            r = wall_v[pl.ds(8688, 16)]
            for k in range(8):
                r = r + g[k] * wall_v[pl.ds(8512 + 16 * k, 16)]
            delta = r[0] - r[1]
            dvec = jnp.zeros((16,), jnp.float32) + delta
            res_v[...] = 1.0 / (1.0 + jnp.exp(dvec))
            pltpu.sync_copy(res_v, out_hbm)


@jax.jit
def _din_sc(idx2d, ut_t, mt_t, ct_t, wall):
    mesh = plsc.VectorSubcoreMesh(core_axis_name="c", subcore_axis_name="s")
    f32 = jnp.float32
    run = functools.partial(
        pl.kernel,
        out_type=jax.ShapeDtypeStruct((16,), f32),
        mesh=mesh,
        scratch_types=[
            pltpu.VMEM((128,), jnp.int32),
            pltpu.VMEM((PASS, K, 128), f32),
            pltpu.VMEM((K, 128), f32),
            pltpu.VMEM((K,), f32),
            pltpu.VMEM_SHARED((NW + 2, K), f32),
            pltpu.VMEM((NW + 2, K), f32),
            pltpu.VMEM((544 * 16,), f32),
            pltpu.VMEM((16,), f32),
            pltpu.SemaphoreType.DMA,
            pltpu.SemaphoreType.DMA,
        ],
        compiler_params=pltpu.CompilerParams(use_tc_tiling_on_sc=True),
    )(_din_body)
    return run(idx2d, ut_t, mt_t, ct_t, wall)


def kernel(user_id, movie_id, movie_cate, user_rate, user_table, movie_table,
           movie_cate_table, W0, b0, W1, b1, W2, b2):
    i32 = jnp.int32
    cat = jnp.concatenate([movie_cate, user_rate]).astype(i32).reshape(NW, VALID)
    idx2d = jnp.pad(cat, ((0, 0), (0, 128 - VALID)))
    um = jnp.concatenate([user_id.astype(i32), movie_id.astype(i32)])
    idx2d = lax.dynamic_update_slice(
        idx2d, jnp.broadcast_to(um, (NW, 2)), (0, 32))
    wall = jnp.concatenate([
        jnp.pad(W0, ((0, 0), (0, 12))).reshape(512, 16),
        jnp.pad(W1, ((0, 0), (0, 8))),
        jnp.pad(W2, ((0, 0), (0, 14))),
        jnp.pad(b0, (0, 12)).reshape(2, 16),
        jnp.pad(b1, (0, 8)).reshape(1, 16),
        jnp.pad(b2, (0, 14)).reshape(1, 16),
    ]).reshape(-1)
    out = _din_sc(idx2d, user_table.T, movie_table.T, movie_cate_table.T,
                  wall)
    return out[1:2]


# speedup vs baseline: 1.1874x; 1.0160x over previous
import functools

import jax
import jax.numpy as jnp
import numpy as np
from jax import lax
from jax.experimental import pallas as pl
from jax.experimental.pallas import tpu as pltpu
from jax.experimental.pallas import tpu_sc as plsc

K = 64
HIST = 200
NW = 16
VALID = 25
PASS = 12


def _din_body(idx_hbm, ut_hbm, mt_hbm, ct_hbm, wall_hbm,
              out_hbm,
              idx_v, blks, abuf, stage_v, shared, part_v,
              wall_v, res_v, gsem, absem):
    cid = lax.axis_index("c")
    sid = lax.axis_index("s")

    @pl.when(cid == 0)
    def _core0():
        pltpu.sync_copy(idx_hbm.at[sid], idx_v)
        i0 = idx_v[pl.ds(0, 16)]
        i1 = idx_v[pl.ds(16, 16)]
        lane16 = lax.iota(jnp.int32, 16)
        cm0 = i0 & (-16)
        cm1 = i1 & (-16)
        l0 = i0 & 15
        l1 = i1 & 15
        cms = [cm0[l] for l in range(16)] + [cm1[l] for l in range(VALID - 16)]
        lns = [l0[l] for l in range(16)] + [l1[l] for l in range(VALID - 16)]
        base = [pl.multiple_of(c & (-128), 128) for c in cms]
        off = [cms[j] - (cms[j] & (-128)) for j in range(VALID)]

        i2 = idx_v[pl.ds(32, 16)]
        ab_id = jnp.where(sid == 1, i2[0], i2[1])
        ab_base = pl.multiple_of(ab_id & (-128), 128)
        ab_off = (ab_id & (-16)) - (ab_id & (-128))

        @pl.when(sid == 1)
        def _fetch_a():
            pltpu.async_copy(ut_hbm.at[:, pl.ds(ab_base, 128)], abuf, absem)

        @pl.when(sid == 2)
        def _fetch_b():
            pltpu.async_copy(mt_hbm.at[:, pl.ds(ab_base, 128)], abuf, absem)

        @pl.when(sid == 0)
        def _prefetch_w():
            pltpu.sync_copy(wall_hbm, wall_v)

        chunks = [jnp.zeros((16,), jnp.float32) for _ in range(4)]
        fmasks = [lane16 == m for m in range(16)]

        def start(j):
            return pltpu.async_copy(
                ct_hbm.at[:, pl.ds(base[j], 128)],
                blks.at[j % PASS], gsem)

        cps = [start(j) for j in range(PASS)]
        for j in range(VALID):
            cps[j].wait()
            lvec = jnp.zeros((16,), jnp.int32) + lns[j]
            for f in range(K):
                v = blks[j % PASS, f, pl.ds(off[j], 16)]
                val = v[lvec]
                i = f // 16
                chunks[i] = chunks[i] + jnp.where(fmasks[f - 16 * i], val, 0.0)
            if j + PASS < VALID:
                cps.append(start(j + PASS))
        for i in range(4):
            stage_v[pl.ds(16 * i, 16)] = chunks[i]
        pltpu.sync_copy(stage_v, shared.at[sid])

        @pl.when((sid == 1) | (sid == 2))
        def _extract_ab():
            pltpu.make_async_copy(ut_hbm.at[:, pl.ds(ab_base, 128)],
                                  abuf, absem).wait()
            ch = [jnp.zeros((16,), jnp.float32) for _ in range(4)]
            ab_lvec = jnp.zeros((16,), jnp.int32) + (ab_id & 15)
            for f in range(K):
                v = abuf[f, pl.ds(ab_off, 16)]
                val = v[ab_lvec]
                i = f // 16
                ch[i] = ch[i] + jnp.where(lane16 == (f - 16 * i), val, 0.0)
            for i in range(4):
                stage_v[pl.ds(16 * i, 16)] = ch[i]
            pltpu.sync_copy(stage_v, shared.at[15 + sid])

        plsc.subcore_barrier()

        @pl.when(sid == 0)
        def _finish():
            pltpu.sync_copy(shared, part_v)
            inv = 1.0 / float(HIST)
            fea = []
            for i in range(4):
                fea.append(part_v[16, pl.ds(16 * i, 16)])
            for i in range(4):
                fea.append(part_v[17, pl.ds(16 * i, 16)])
            for seg in range(2):
                for i in range(4):
                    s = part_v[8 * seg, pl.ds(16 * i, 16)]
                    for j in range(1, 8):
                        s = s + part_v[8 * seg + j, pl.ds(16 * i, 16)]
                    fea.append(s * inv)

            h0 = wall_v[pl.ds(8640, 16)]
            h1 = wall_v[pl.ds(8656, 16)]
            for t in range(16):
                chunk = fea[t]
                for l in range(16):
                    f = chunk[l]
                    k = 16 * t + l
                    h0 = h0 + f * wall_v[pl.ds(32 * k, 16)]
                    h1 = h1 + f * wall_v[pl.ds(32 * k + 16, 16)]
            h0 = jnp.maximum(h0, 0.0)
            h1 = jnp.maximum(h1, 0.0)

            g = wall_v[pl.ds(8672, 16)]
            for k in range(16):
                g = g + h0[k] * wall_v[pl.ds(8192 + 16 * k, 16)]
            for k in range(4):
                g = g + h1[k] * wall_v[pl.ds(8192 + 16 * (16 + k), 16)]
            g = jnp.maximum(g, 0.0)

            r = wall_v[pl.ds(8688, 16)]
            for k in range(8):
                r = r + g[k] * wall_v[pl.ds(8512 + 16 * k, 16)]
            delta = r[0] - r[1]
            dvec = jnp.zeros((16,), jnp.float32) + delta
            res_v[...] = 1.0 / (1.0 + jnp.exp(dvec))
            pltpu.sync_copy(res_v, out_hbm)


@jax.jit
def _din_sc(idx2d, ut_t, mt_t, ct_t, wall):
    mesh = plsc.VectorSubcoreMesh(core_axis_name="c", subcore_axis_name="s")
    f32 = jnp.float32
    run = functools.partial(
        pl.kernel,
        out_type=jax.ShapeDtypeStruct((16,), f32),
        mesh=mesh,
        scratch_types=[
            pltpu.VMEM((128,), jnp.int32),
            pltpu.VMEM((PASS, K, 128), f32),
            pltpu.VMEM((K, 128), f32),
            pltpu.VMEM((K,), f32),
            pltpu.VMEM_SHARED((NW + 2, K), f32),
            pltpu.VMEM((NW + 2, K), f32),
            pltpu.VMEM((544 * 16,), f32),
            pltpu.VMEM((16,), f32),
            pltpu.SemaphoreType.DMA,
            pltpu.SemaphoreType.DMA,
        ],
        compiler_params=pltpu.CompilerParams(use_tc_tiling_on_sc=True),
    )(_din_body)
    return run(idx2d, ut_t, mt_t, ct_t, wall)


def kernel(user_id, movie_id, movie_cate, user_rate, user_table, movie_table,
           movie_cate_table, W0, b0, W1, b1, W2, b2):
    i32 = jnp.int32
    cat = jnp.concatenate([movie_cate, user_rate]).astype(i32).reshape(NW, VALID)
    idx2d = jnp.pad(cat, ((0, 0), (0, 128 - VALID)))
    um = jnp.concatenate([user_id.astype(i32), movie_id.astype(i32)])
    idx2d = lax.dynamic_update_slice(
        idx2d, jnp.broadcast_to(um, (NW, 2)), (0, 32))
    wall = jnp.concatenate([
        jnp.pad(W0, ((0, 0), (0, 12))).reshape(512, 16),
        jnp.pad(W1, ((0, 0), (0, 8))),
        jnp.pad(W2, ((0, 0), (0, 14))),
        jnp.pad(b0, (0, 12)).reshape(2, 16),
        jnp.pad(b1, (0, 8)).reshape(1, 16),
        jnp.pad(b2, (0, 14)).reshape(1, 16),
    ]).reshape(-1)
    out = _din_sc(idx2d, user_table.T, movie_table.T, movie_cate_table.T,
                  wall)
    return out[1:2]
